# baseline (speedup 1.0000x reference)
import functools

import jax
import jax.numpy as jnp
from jax import lax
from jax.experimental import pallas as pl
from jax.experimental.pallas import tpu as pltpu
from jax.experimental.pallas import tpu_sc as plsc

NN = 10000
EE = 320000
NC = 2
NS = 16
NW = NC * NS
BLK = 128
NB = 80
KR = 8
NP = 10112
STRIPE = NP // NS
TRASH = NN

_MESH = plsc.VectorSubcoreMesh(
    core_axis_name="c", subcore_axis_name="s", num_cores=NC, num_subcores=NS)


@functools.cache
def _sc_segment_sum(F):
  def body(g_hbm, srcp_hbm, dstp_hbm, zer_hbm, out_hbm, src_v, dst_v, *rest):
    rows = list(rest[:2])
    gsem = list(rest[2:4])
    acc_sh = rest[4]
    cid = lax.axis_index("c")
    sid = lax.axis_index("s")
    wid = cid * NS + sid
    r0 = sid * STRIPE
    pltpu.sync_copy(zer_hbm, acc_sh.at[pl.ds(r0, STRIPE)])
    pltpu.sync_copy(srcp_hbm.at[wid], src_v)
    pltpu.sync_copy(dstp_hbm.at[wid], dst_v)
    plsc.subcore_barrier()

    pltpu.async_copy(g_hbm.at[src_v.at[0]], rows[0], gsem[0])

    def blk2(m, carry):
      for k in range(2):
        j = 2 * m + k
        pltpu.make_async_copy(g_hbm.at[src_v.at[j]], rows[k], gsem[k]).wait()

        @pl.when(j + 1 < NB)
        def _():
          pltpu.async_copy(g_hbm.at[src_v.at[j + 1]], rows[1 - k],
                           gsem[1 - k])

        pltpu.sync_copy(rows[k], acc_sh.at[dst_v.at[j]], add=True)
      return carry

    lax.fori_loop(0, NB // 2, blk2, 0)
    plsc.subcore_barrier()
    pltpu.sync_copy(acc_sh.at[pl.ds(r0, STRIPE)],
                    out_hbm.at[cid, pl.ds(r0, STRIPE)])

  return pl.kernel(
      body,
      out_type=jax.ShapeDtypeStruct((NC, NP, F), jnp.float32),
      mesh=_MESH,
      compiler_params=pltpu.CompilerParams(use_tc_tiling_on_sc=False),
      scratch_types=(
          [pltpu.VMEM((NB, BLK), jnp.int32),
           pltpu.VMEM((NB, BLK), jnp.int32)]
          + [pltpu.VMEM((BLK, F), jnp.float32) for _ in range(2)]
          + [pltpu.SemaphoreType.DMA for _ in range(2)]
          + [pltpu.VMEM_SHARED((NP, F), jnp.float32)]
      ),
  )


@functools.cache
def _sc_degree():
  F = 16

  def body(ones_hbm, dstp_hbm, zer_hbm, out_hbm, ones_v, dst_v, sem, acc_sh):
    cid = lax.axis_index("c")
    sid = lax.axis_index("s")
    wid = cid * NS + sid
    r0 = sid * STRIPE
    pltpu.sync_copy(zer_hbm, acc_sh.at[pl.ds(r0, STRIPE)])
    pltpu.sync_copy(ones_hbm, ones_v)
    pltpu.sync_copy(dstp_hbm.at[wid], dst_v)
    plsc.subcore_barrier()

    def blk(j, carry):
      pltpu.sync_copy(ones_v, acc_sh.at[dst_v.at[j]], add=True)
      return carry

    lax.fori_loop(0, NB, blk, 0)
    plsc.subcore_barrier()
    pltpu.sync_copy(acc_sh.at[pl.ds(r0, STRIPE)],
                    out_hbm.at[cid, pl.ds(r0, STRIPE)])

  return pl.kernel(
      body,
      out_type=jax.ShapeDtypeStruct((NC, NP, F), jnp.float32),
      mesh=_MESH,
      compiler_params=pltpu.CompilerParams(use_tc_tiling_on_sc=False),
      scratch_types=[
          pltpu.VMEM((BLK, F), jnp.float32),
          pltpu.VMEM((NB, BLK), jnp.int32),
          pltpu.SemaphoreType.DMA,
          pltpu.VMEM_SHARED((NP, F), jnp.float32),
      ],
  )


def _dinv_body(p_ref, o_ref):
  o_ref[...] = lax.rsqrt(p_ref[0] + p_ref[1] + 1.0)


def _dinv(deg2):
  return pl.pallas_call(
      _dinv_body,
      out_shape=jax.ShapeDtypeStruct((NP // BLK, BLK), jnp.float32),
  )(deg2)


_RB = 2000


def _lin_body(x_ref, w_ref, dinv_ref, o_ref):
  o_ref[...] = dinv_ref[...] * jnp.dot(
      x_ref[...], w_ref[...], preferred_element_type=jnp.float32)


def _lin(x, w, dinv):
  di, do = w.shape
  return pl.pallas_call(
      _lin_body,
      grid=(NN // _RB,),
      in_specs=[
          pl.BlockSpec((_RB, di), lambda k: (k, 0)),
          pl.BlockSpec((di, do), lambda k: (0, 0)),
          pl.BlockSpec((_RB, 1), lambda k: (k, 0)),
      ],
      out_specs=pl.BlockSpec((_RB, do), lambda k: (k, 0)),
      out_shape=jax.ShapeDtypeStruct((NN, do), jnp.float32),
  )(x, w, dinv)


def _combine_body(relu, nxt, p0_ref, p1_ref, g_ref, dinv_ref, b_ref, *rest):
  h = dinv_ref[...] * (p0_ref[...] + p1_ref[...] + g_ref[...]) + b_ref[...]
  if relu:
    h = jnp.maximum(h, 0.0)
  if nxt:
    w_ref, o_ref = rest
    o_ref[...] = dinv_ref[...] * jnp.dot(
        h, w_ref[...], preferred_element_type=jnp.float32)
  else:
    (o_ref,) = rest
    o_ref[...] = h


def _combine(p0, p1, g, dinv, b, w=None, relu=True):
  F = g.shape[1]
  specs = [
      pl.BlockSpec((_RB, F), lambda k: (k, 0)),
      pl.BlockSpec((_RB, F), lambda k: (k, 0)),
      pl.BlockSpec((_RB, F), lambda k: (k, 0)),
      pl.BlockSpec((_RB, 1), lambda k: (k, 0)),
      pl.BlockSpec((1, F), lambda k: (0, 0)),
  ]
  args = [p0, p1, g, dinv, b.reshape(1, F)]
  Fo = F
  if w is not None:
    Fo = w.shape[1]
    specs.append(pl.BlockSpec((F, Fo), lambda k: (0, 0)))
    args.append(w)
  return pl.pallas_call(
      functools.partial(_combine_body, relu, w is not None),
      grid=(NN // _RB,),
      in_specs=specs,
      out_specs=pl.BlockSpec((_RB, Fo), lambda k: (k, 0)),
      out_shape=jax.ShapeDtypeStruct((NN, Fo), jnp.float32),
  )(*args)


_KB = 2000
_CB = 3200


def _enc_body(h_ref, we_ref, be_ref, o_ref, acc_ref):
  k = pl.program_id(0)

  @pl.when(k == 0)
  def _():
    acc_ref[...] = jnp.zeros_like(acc_ref)

  acc_ref[...] += jnp.sum(we_ref[...] * h_ref[...], axis=0, keepdims=True)

  @pl.when(k == pl.num_programs(0) - 1)
  def _():
    o_ref[...] = acc_ref[...] + be_ref[...]


def _encoder(h3f, we, be):
  L = we.shape[1]
  return pl.pallas_call(
      _enc_body,
      grid=(h3f.shape[0] // _KB,),
      in_specs=[
          pl.BlockSpec((_KB, 1), lambda k: (k, 0)),
          pl.BlockSpec((_KB, L), lambda k: (k, 0)),
          pl.BlockSpec((1, L), lambda k: (0, 0)),
      ],
      out_specs=pl.BlockSpec((1, L), lambda k: (0, 0)),
      out_shape=jax.ShapeDtypeStruct((1, L), jnp.float32),
      scratch_shapes=[pltpu.VMEM((1, L), jnp.float32)],
  )(h3f, we, be.reshape(1, L))


def _dec_body(z_ref, wd_ref, bd_ref, o_ref):
  o_ref[...] = jnp.sum(z_ref[...] * wd_ref[...], axis=0,
                       keepdims=True) + bd_ref[...]


def _decoder(zc, wd, bd):
  L, M = wd.shape
  return pl.pallas_call(
      _dec_body,
      grid=(M // _CB,),
      in_specs=[
          pl.BlockSpec((L, 1), lambda k: (0, 0)),
          pl.BlockSpec((L, _CB), lambda k: (0, k)),
          pl.BlockSpec((1, _CB), lambda k: (0, k)),
      ],
      out_specs=pl.BlockSpec((1, _CB), lambda k: (0, k)),
      out_shape=jax.ShapeDtypeStruct((1, M), jnp.float32),
  )(zc, wd, bd.reshape(1, M))


def kernel(x, edge_index, batch_size, batch_index, W1, b1, W2, b2, W3, b3,
           We, be, Wd, bd, W4, b4, W5, b5, W6, b6):
  f32 = jnp.float32
  pad = NW * NB * BLK - EE
  srcp = jnp.concatenate(
      [edge_index[0], jnp.zeros((pad,), jnp.int32)]).reshape(NW, NB, BLK)
  dstp = jnp.concatenate(
      [edge_index[1], jnp.full((pad,), TRASH, jnp.int32)]).reshape(NW, NB, BLK)

  def seg(g):
    F = g.shape[1]
    s = _sc_segment_sum(F)(g, srcp, dstp, jnp.zeros((STRIPE, F), f32))
    return s[0, :NN], s[1, :NN]

  degp = _sc_degree()(jnp.ones((BLK, 16), f32), dstp,
                      jnp.zeros((STRIPE, 16), f32))
  deg2 = degp[:, :, 0].reshape(NC, NP // BLK, BLK)
  dinv = _dinv(deg2).reshape(NP, 1)[:NN]

  g1 = _lin(x, W1, dinv)
  s0, s1 = seg(g1)
  g2 = _combine(s0, s1, g1, dinv, b1, W2)
  s0, s1 = seg(g2)
  g3 = _combine(s0, s1, g2, dinv, b2, W3)
  s0, s1 = seg(g3)
  h3 = _combine(s0, s1, g3, dinv, b3, relu=False)

  z = _encoder(h3.reshape(NN * 16, 1), We, be)
  h4f = _decoder(z.reshape(We.shape[1], 1), Wd, bd)
  h4 = h4f.reshape(NN, 16)

  g4 = _lin(h4, W4, dinv)
  s0, s1 = seg(g4)
  g5 = _combine(s0, s1, g4, dinv, b4, W5)
  s0, s1 = seg(g5)
  W6p = jnp.concatenate([W6, jnp.zeros((W6.shape[0], 15), f32)], axis=1)
  b6p = jnp.concatenate([b6, jnp.zeros((15,), f32)])
  g6 = _combine(s0, s1, g5, dinv, b5, W6p)
  s0, s1 = seg(g6)
  out = _combine(s0, s1, g6, dinv, b6p, relu=False)
  return out[:, :1].reshape(1, NN)


# speedup vs baseline: 1.2711x; 1.2711x over previous
import functools

import jax
import jax.numpy as jnp
from jax import lax
from jax.experimental import pallas as pl
from jax.experimental.pallas import tpu as pltpu
from jax.experimental.pallas import tpu_sc as plsc

NN = 10000
EE = 320000
NC = 2
NS = 16
NW = NC * NS
BLK = 128
NB = 79
NP = 10112
STRIPE = NP // NS
TRASH = NN

_MESH = plsc.VectorSubcoreMesh(
    core_axis_name="c", subcore_axis_name="s", num_cores=NC, num_subcores=NS)


@functools.cache
def _sc_segment_sum(F):
  def body(g_hbm, srcp_hbm, dstp_hbm, zer_hbm, out_hbm, src_v, dst_v, rows_v,
           sem, acc_sh):
    cid = lax.axis_index("c")
    sid = lax.axis_index("s")
    wid = cid * NS + sid
    r0 = sid * STRIPE
    pltpu.sync_copy(zer_hbm, acc_sh.at[pl.ds(r0, STRIPE)])
    pltpu.sync_copy(srcp_hbm.at[wid], src_v)
    pltpu.sync_copy(dstp_hbm.at[wid], dst_v)
    plsc.subcore_barrier()

    def blk(j, carry):
      pltpu.async_copy(g_hbm.at[src_v.at[j]], rows_v, sem).wait()
      pltpu.sync_copy(rows_v, acc_sh.at[dst_v.at[j]], add=True)
      return carry

    lax.fori_loop(0, NB, blk, 0)
    plsc.subcore_barrier()
    pltpu.sync_copy(acc_sh.at[pl.ds(r0, STRIPE)],
                    out_hbm.at[cid, pl.ds(r0, STRIPE)])

  return pl.kernel(
      body,
      out_type=jax.ShapeDtypeStruct((NC, NP, F), jnp.float32),
      mesh=_MESH,
      compiler_params=pltpu.CompilerParams(use_tc_tiling_on_sc=False),
      scratch_types=[
          pltpu.VMEM((NB, BLK), jnp.int32),
          pltpu.VMEM((NB, BLK), jnp.int32),
          pltpu.VMEM((BLK, F), jnp.float32),
          pltpu.SemaphoreType.DMA,
          pltpu.VMEM_SHARED((NP, F), jnp.float32),
      ],
  )


@functools.cache
def _sc_degree():
  F = 16

  def body(ones_hbm, dstp_hbm, zer_hbm, out_hbm, ones_v, dst_v, sem, acc_sh):
    cid = lax.axis_index("c")
    sid = lax.axis_index("s")
    wid = cid * NS + sid
    r0 = sid * STRIPE
    pltpu.sync_copy(zer_hbm, acc_sh.at[pl.ds(r0, STRIPE)])
    pltpu.sync_copy(ones_hbm, ones_v)
    pltpu.sync_copy(dstp_hbm.at[wid], dst_v)
    plsc.subcore_barrier()

    def blk(j, carry):
      pltpu.sync_copy(ones_v, acc_sh.at[dst_v.at[j]], add=True)
      return carry

    lax.fori_loop(0, NB, blk, 0)
    plsc.subcore_barrier()
    pltpu.sync_copy(acc_sh.at[pl.ds(r0, STRIPE)],
                    out_hbm.at[cid, pl.ds(r0, STRIPE)])

  return pl.kernel(
      body,
      out_type=jax.ShapeDtypeStruct((NC, NP, F), jnp.float32),
      mesh=_MESH,
      compiler_params=pltpu.CompilerParams(use_tc_tiling_on_sc=False),
      scratch_types=[
          pltpu.VMEM((BLK, F), jnp.float32),
          pltpu.VMEM((NB, BLK), jnp.int32),
          pltpu.SemaphoreType.DMA,
          pltpu.VMEM_SHARED((NP, F), jnp.float32),
      ],
  )


def _dinv_body(p_ref, o_ref):
  o_ref[...] = lax.rsqrt(p_ref[0] + p_ref[1] + 1.0)


def _dinv(deg2):
  return pl.pallas_call(
      _dinv_body,
      out_shape=jax.ShapeDtypeStruct((NP // BLK, BLK), jnp.float32),
  )(deg2)


_RB = 2000


def _lin_body(x_ref, w_ref, dinv_ref, o_ref):
  o_ref[...] = dinv_ref[...] * jnp.dot(
      x_ref[...], w_ref[...], preferred_element_type=jnp.float32)


def _lin(x, w, dinv):
  di, do = w.shape
  return pl.pallas_call(
      _lin_body,
      grid=(NN // _RB,),
      in_specs=[
          pl.BlockSpec((_RB, di), lambda k: (k, 0)),
          pl.BlockSpec((di, do), lambda k: (0, 0)),
          pl.BlockSpec((_RB, 1), lambda k: (k, 0)),
      ],
      out_specs=pl.BlockSpec((_RB, do), lambda k: (k, 0)),
      out_shape=jax.ShapeDtypeStruct((NN, do), jnp.float32),
  )(x, w, dinv)


def _combine_body(relu, nxt, p0_ref, p1_ref, g_ref, dinv_ref, b_ref, *rest):
  h = dinv_ref[...] * (p0_ref[...] + p1_ref[...] + g_ref[...]) + b_ref[...]
  if relu:
    h = jnp.maximum(h, 0.0)
  if nxt:
    w_ref, o_ref = rest
    o_ref[...] = dinv_ref[...] * jnp.dot(
        h, w_ref[...], preferred_element_type=jnp.float32)
  else:
    (o_ref,) = rest
    o_ref[...] = h


def _combine(p0, p1, g, dinv, b, w=None, relu=True):
  F = g.shape[1]
  specs = [
      pl.BlockSpec((_RB, F), lambda k: (k, 0)),
      pl.BlockSpec((_RB, F), lambda k: (k, 0)),
      pl.BlockSpec((_RB, F), lambda k: (k, 0)),
      pl.BlockSpec((_RB, 1), lambda k: (k, 0)),
      pl.BlockSpec((1, F), lambda k: (0, 0)),
  ]
  args = [p0, p1, g, dinv, b.reshape(1, F)]
  Fo = F
  if w is not None:
    Fo = w.shape[1]
    specs.append(pl.BlockSpec((F, Fo), lambda k: (0, 0)))
    args.append(w)
  return pl.pallas_call(
      functools.partial(_combine_body, relu, w is not None),
      grid=(NN // _RB,),
      in_specs=specs,
      out_specs=pl.BlockSpec((_RB, Fo), lambda k: (k, 0)),
      out_shape=jax.ShapeDtypeStruct((NN, Fo), jnp.float32),
  )(*args)


_KB = 2000
_CB = 3200


def _enc_body(h_ref, we_ref, be_ref, o_ref, acc_ref):
  k = pl.program_id(0)

  @pl.when(k == 0)
  def _():
    acc_ref[...] = jnp.zeros_like(acc_ref)

  acc_ref[...] += jnp.sum(we_ref[...] * h_ref[...], axis=0, keepdims=True)

  @pl.when(k == pl.num_programs(0) - 1)
  def _():
    o_ref[...] = acc_ref[...] + be_ref[...]


def _encoder(h3f, we, be):
  L = we.shape[1]
  return pl.pallas_call(
      _enc_body,
      grid=(h3f.shape[0] // _KB,),
      in_specs=[
          pl.BlockSpec((_KB, 1), lambda k: (k, 0)),
          pl.BlockSpec((_KB, L), lambda k: (k, 0)),
          pl.BlockSpec((1, L), lambda k: (0, 0)),
      ],
      out_specs=pl.BlockSpec((1, L), lambda k: (0, 0)),
      out_shape=jax.ShapeDtypeStruct((1, L), jnp.float32),
      scratch_shapes=[pltpu.VMEM((1, L), jnp.float32)],
  )(h3f, we, be.reshape(1, L))


def _dec_body(z_ref, wd_ref, bd_ref, o_ref):
  o_ref[...] = jnp.sum(z_ref[...] * wd_ref[...], axis=0,
                       keepdims=True) + bd_ref[...]


def _decoder(zc, wd, bd):
  L, M = wd.shape
  return pl.pallas_call(
      _dec_body,
      grid=(M // _CB,),
      in_specs=[
          pl.BlockSpec((L, 1), lambda k: (0, 0)),
          pl.BlockSpec((L, _CB), lambda k: (0, k)),
          pl.BlockSpec((1, _CB), lambda k: (0, k)),
      ],
      out_specs=pl.BlockSpec((1, _CB), lambda k: (0, k)),
      out_shape=jax.ShapeDtypeStruct((1, M), jnp.float32),
  )(zc, wd, bd.reshape(1, M))


def kernel(x, edge_index, batch_size, batch_index, W1, b1, W2, b2, W3, b3,
           We, be, Wd, bd, W4, b4, W5, b5, W6, b6):
  f32 = jnp.float32
  pad = NW * NB * BLK - EE
  srcp = jnp.concatenate(
      [edge_index[0], jnp.zeros((pad,), jnp.int32)]).reshape(NW, NB, BLK)
  dstp = jnp.concatenate(
      [edge_index[1], jnp.full((pad,), TRASH, jnp.int32)]).reshape(NW, NB, BLK)

  def seg(g):
    F = g.shape[1]
    s = _sc_segment_sum(F)(g, srcp, dstp, jnp.zeros((STRIPE, F), f32))
    return s[0, :NN], s[1, :NN]

  degp = _sc_degree()(jnp.ones((BLK, 16), f32), dstp,
                      jnp.zeros((STRIPE, 16), f32))
  deg2 = degp[:, :, 0].reshape(NC, NP // BLK, BLK)
  dinv = _dinv(deg2).reshape(NP, 1)[:NN]

  g1 = _lin(x, W1, dinv)
  s0, s1 = seg(g1)
  g2 = _combine(s0, s1, g1, dinv, b1, W2)
  s0, s1 = seg(g2)
  g3 = _combine(s0, s1, g2, dinv, b2, W3)
  s0, s1 = seg(g3)
  h3 = _combine(s0, s1, g3, dinv, b3, relu=False)

  z = _encoder(h3.reshape(NN * 16, 1), We, be)
  h4f = _decoder(z.reshape(We.shape[1], 1), Wd, bd)
  h4 = h4f.reshape(NN, 16)

  g4 = _lin(h4, W4, dinv)
  s0, s1 = seg(g4)
  g5 = _combine(s0, s1, g4, dinv, b4, W5)
  s0, s1 = seg(g5)
  W6p = jnp.concatenate([W6, jnp.zeros((W6.shape[0], 15), f32)], axis=1)
  b6p = jnp.concatenate([b6, jnp.zeros((15,), f32)])
  g6 = _combine(s0, s1, g5, dinv, b5, W6p)
  s0, s1 = seg(g6)
  out = _combine(s0, s1, g6, dinv, b6p, relu=False)
  return out[:, :1].reshape(1, NN)


# speedup vs baseline: 1.2929x; 1.0172x over previous
import functools

import jax
import jax.numpy as jnp
from jax import lax
from jax.experimental import pallas as pl
from jax.experimental.pallas import tpu as pltpu
from jax.experimental.pallas import tpu_sc as plsc

NN = 10000
EE = 320000
NC = 2
NS = 16
NW = NC * NS
BLK = 128
NB = 79
NP = 10112
STRIPE = NP // NS
TRASH = NN

_MESH = plsc.VectorSubcoreMesh(
    core_axis_name="c", subcore_axis_name="s", num_cores=NC, num_subcores=NS)


@functools.cache
def _sc_segment_sum(F):
  def body(g_hbm, srcp_hbm, dstp_hbm, zer_hbm, out_hbm, src_v, dst_v, rows_v,
           sem, acc_sh):
    cid = lax.axis_index("c")
    sid = lax.axis_index("s")
    wid = cid * NS + sid
    r0 = sid * STRIPE
    pltpu.sync_copy(zer_hbm, acc_sh.at[pl.ds(r0, STRIPE)])
    pltpu.sync_copy(srcp_hbm.at[wid], src_v)
    pltpu.sync_copy(dstp_hbm.at[wid], dst_v)
    plsc.subcore_barrier()

    def blk(j, carry):
      pltpu.async_copy(g_hbm.at[src_v.at[j]], rows_v, sem).wait()
      pltpu.sync_copy(rows_v, acc_sh.at[dst_v.at[j]], add=True)
      return carry

    lax.fori_loop(0, NB, blk, 0)
    plsc.subcore_barrier()
    pltpu.sync_copy(acc_sh.at[pl.ds(r0, STRIPE)],
                    out_hbm.at[cid, pl.ds(r0, STRIPE)])

  return pl.kernel(
      body,
      out_type=jax.ShapeDtypeStruct((NC, NP, F), jnp.float32),
      mesh=_MESH,
      compiler_params=pltpu.CompilerParams(use_tc_tiling_on_sc=False),
      scratch_types=[
          pltpu.VMEM((NB, BLK), jnp.int32),
          pltpu.VMEM((NB, BLK), jnp.int32),
          pltpu.VMEM((BLK, F), jnp.float32),
          pltpu.SemaphoreType.DMA,
          pltpu.VMEM_SHARED((NP, F), jnp.float32),
      ],
  )


@functools.cache
def _sc_degree():
  F = 16

  def body(ones_hbm, dstp_hbm, zer_hbm, out_hbm, ones_v, dst_v, sem, acc_sh):
    cid = lax.axis_index("c")
    sid = lax.axis_index("s")
    wid = cid * NS + sid
    r0 = sid * STRIPE
    pltpu.sync_copy(zer_hbm, acc_sh.at[pl.ds(r0, STRIPE)])
    pltpu.sync_copy(ones_hbm, ones_v)
    pltpu.sync_copy(dstp_hbm.at[wid], dst_v)
    plsc.subcore_barrier()

    def blk(j, carry):
      pltpu.sync_copy(ones_v, acc_sh.at[dst_v.at[j]], add=True)
      return carry

    lax.fori_loop(0, NB, blk, 0)
    plsc.subcore_barrier()
    pltpu.sync_copy(acc_sh.at[pl.ds(r0, STRIPE)],
                    out_hbm.at[cid, pl.ds(r0, STRIPE)])

  return pl.kernel(
      body,
      out_type=jax.ShapeDtypeStruct((NC, NP, F), jnp.float32),
      mesh=_MESH,
      compiler_params=pltpu.CompilerParams(use_tc_tiling_on_sc=False),
      scratch_types=[
          pltpu.VMEM((BLK, F), jnp.float32),
          pltpu.VMEM((NB, BLK), jnp.int32),
          pltpu.SemaphoreType.DMA,
          pltpu.VMEM_SHARED((NP, F), jnp.float32),
      ],
  )


_RB = 2000


def _dinv_of(deg_blk):
  return lax.rsqrt(deg_blk[0, :, 0:1] + deg_blk[1, :, 0:1] + 1.0)


def _lin_body(x_ref, w_ref, deg_ref, o_ref):
  d = _dinv_of(deg_ref[...])
  o_ref[...] = d * jnp.dot(
      x_ref[...], w_ref[...], preferred_element_type=jnp.float32)


def _lin(x, w, degp):
  di, do = w.shape
  return pl.pallas_call(
      _lin_body,
      grid=(NN // _RB,),
      in_specs=[
          pl.BlockSpec((_RB, di), lambda k: (k, 0)),
          pl.BlockSpec((di, do), lambda k: (0, 0)),
          pl.BlockSpec((2, _RB, 16), lambda k: (0, k, 0)),
      ],
      out_specs=pl.BlockSpec((_RB, do), lambda k: (k, 0)),
      out_shape=jax.ShapeDtypeStruct((NN, do), jnp.float32),
  )(x, w, degp)


def _combine_body(relu, nxt, s_ref, g_ref, deg_ref, b_ref, *rest):
  d = _dinv_of(deg_ref[...])
  s = s_ref[...]
  h = d * (s[0] + s[1] + g_ref[...]) + b_ref[...]
  if relu:
    h = jnp.maximum(h, 0.0)
  if nxt:
    w_ref, o_ref = rest
    o_ref[...] = d * jnp.dot(h, w_ref[...],
                             preferred_element_type=jnp.float32)
  else:
    (o_ref,) = rest
    o_ref[...] = h


def _combine(s, g, degp, b, w=None, relu=True):
  F = g.shape[1]
  specs = [
      pl.BlockSpec((2, _RB, F), lambda k: (0, k, 0)),
      pl.BlockSpec((_RB, F), lambda k: (k, 0)),
      pl.BlockSpec((2, _RB, 16), lambda k: (0, k, 0)),
      pl.BlockSpec((1, F), lambda k: (0, 0)),
  ]
  args = [s, g, degp, b.reshape(1, F)]
  Fo = F
  if w is not None:
    Fo = w.shape[1]
    specs.append(pl.BlockSpec((F, Fo), lambda k: (0, 0)))
    args.append(w)
  return pl.pallas_call(
      functools.partial(_combine_body, relu, w is not None),
      grid=(NN // _RB,),
      in_specs=specs,
      out_specs=pl.BlockSpec((_RB, Fo), lambda k: (k, 0)),
      out_shape=jax.ShapeDtypeStruct((NN, Fo), jnp.float32),
  )(*args)


_KB = 2000
_CB = 3200


def _enc_body(h_ref, we_ref, be_ref, o_ref, acc_ref):
  k = pl.program_id(0)

  @pl.when(k == 0)
  def _():
    acc_ref[...] = jnp.zeros_like(acc_ref)

  acc_ref[...] += jnp.sum(we_ref[...] * h_ref[...], axis=0, keepdims=True)

  @pl.when(k == pl.num_programs(0) - 1)
  def _():
    o_ref[...] = acc_ref[...] + be_ref[...]


def _encoder(h3f, we, be):
  L = we.shape[1]
  return pl.pallas_call(
      _enc_body,
      grid=(h3f.shape[0] // _KB,),
      in_specs=[
          pl.BlockSpec((_KB, 1), lambda k: (k, 0)),
          pl.BlockSpec((_KB, L), lambda k: (k, 0)),
          pl.BlockSpec((1, L), lambda k: (0, 0)),
      ],
      out_specs=pl.BlockSpec((1, L), lambda k: (0, 0)),
      out_shape=jax.ShapeDtypeStruct((1, L), jnp.float32),
      scratch_shapes=[pltpu.VMEM((1, L), jnp.float32)],
  )(h3f, we, be.reshape(1, L))


def _dec_body(z_ref, wd_ref, bd_ref, o_ref):
  o_ref[...] = jnp.sum(z_ref[...] * wd_ref[...], axis=0,
                       keepdims=True) + bd_ref[...]


def _decoder(zc, wd, bd):
  L, M = wd.shape
  return pl.pallas_call(
      _dec_body,
      grid=(M // _CB,),
      in_specs=[
          pl.BlockSpec((L, 1), lambda k: (0, 0)),
          pl.BlockSpec((L, _CB), lambda k: (0, k)),
          pl.BlockSpec((1, _CB), lambda k: (0, k)),
      ],
      out_specs=pl.BlockSpec((1, _CB), lambda k: (0, k)),
      out_shape=jax.ShapeDtypeStruct((1, M), jnp.float32),
  )(zc, wd, bd.reshape(1, M))


def kernel(x, edge_index, batch_size, batch_index, W1, b1, W2, b2, W3, b3,
           We, be, Wd, bd, W4, b4, W5, b5, W6, b6):
  f32 = jnp.float32
  pad = NW * NB * BLK - EE
  srcp = jnp.concatenate(
      [edge_index[0], jnp.zeros((pad,), jnp.int32)]).reshape(NW, NB, BLK)
  dstp = jnp.concatenate(
      [edge_index[1], jnp.full((pad,), TRASH, jnp.int32)]).reshape(NW, NB, BLK)

  def seg(g):
    F = g.shape[1]
    return _sc_segment_sum(F)(g, srcp, dstp, jnp.zeros((STRIPE, F), f32))

  degp = _sc_degree()(jnp.ones((BLK, 16), f32), dstp,
                      jnp.zeros((STRIPE, 16), f32))

  g1 = _lin(x, W1, degp)
  g2 = _combine(seg(g1), g1, degp, b1, W2)
  g3 = _combine(seg(g2), g2, degp, b2, W3)
  h3 = _combine(seg(g3), g3, degp, b3, relu=False)

  z = _encoder(h3.reshape(NN * 16, 1), We, be)
  h4f = _decoder(z.reshape(We.shape[1], 1), Wd, bd)
  h4 = h4f.reshape(NN, 16)

  g4 = _lin(h4, W4, degp)
  g5 = _combine(seg(g4), g4, degp, b4, W5)
  W6p = jnp.concatenate([W6, jnp.zeros((W6.shape[0], 15), f32)], axis=1)
  b6p = jnp.concatenate([b6, jnp.zeros((15,), f32)])
  g6 = _combine(seg(g5), g5, degp, b5, W6p)
  out = _combine(seg(g6), g6, degp, b6p, relu=False)
  return out[:, :1].reshape(1, NN)


# speedup vs baseline: 1.3667x; 1.0570x over previous
import functools

import jax
import jax.numpy as jnp
from jax import lax
from jax.experimental import pallas as pl
from jax.experimental.pallas import tpu as pltpu
from jax.experimental.pallas import tpu_sc as plsc

NN = 10000
EE = 320000
NC = 2
NS = 16
NW = NC * NS
BLK = 128
NB = 79
NP = 10112
STRIPE = NP // NS
TRASH = NN

_MESH = plsc.VectorSubcoreMesh(
    core_axis_name="c", subcore_axis_name="s", num_cores=NC, num_subcores=NS)


@functools.cache
def _sc_segment_sum(F):
  def body(g_hbm, srcp_hbm, dstp_hbm, zer_hbm, out_hbm, src_v, dst_v, rows_v,
           sem, acc_sh):
    cid = lax.axis_index("c")
    sid = lax.axis_index("s")
    wid = cid * NS + sid
    r0 = sid * STRIPE
    pltpu.sync_copy(zer_hbm, acc_sh.at[pl.ds(r0, STRIPE)])
    pltpu.sync_copy(srcp_hbm.at[wid], src_v)
    pltpu.sync_copy(dstp_hbm.at[wid], dst_v)
    plsc.subcore_barrier()

    def blk(j, carry):
      pltpu.async_copy(g_hbm.at[src_v.at[j]], rows_v, sem).wait()
      pltpu.sync_copy(rows_v, acc_sh.at[dst_v.at[j]], add=True)
      return carry

    lax.fori_loop(0, NB, blk, 0)
    plsc.subcore_barrier()
    pltpu.sync_copy(acc_sh.at[pl.ds(r0, STRIPE)],
                    out_hbm.at[cid, pl.ds(r0, STRIPE)])

  return pl.kernel(
      body,
      out_type=jax.ShapeDtypeStruct((NC, NP, F), jnp.float32),
      mesh=_MESH,
      compiler_params=pltpu.CompilerParams(use_tc_tiling_on_sc=False),
      scratch_types=[
          pltpu.VMEM((NB, BLK), jnp.int32),
          pltpu.VMEM((NB, BLK), jnp.int32),
          pltpu.VMEM((BLK, F), jnp.float32),
          pltpu.SemaphoreType.DMA,
          pltpu.VMEM_SHARED((NP, F), jnp.float32),
      ],
  )


@functools.cache
def _sc_degree():
  F = 16

  def body(ones_hbm, dstp_hbm, zer_hbm, out_hbm, ones_v, dst_v, sem, acc_sh):
    cid = lax.axis_index("c")
    sid = lax.axis_index("s")
    wid = cid * NS + sid
    r0 = sid * STRIPE
    pltpu.sync_copy(zer_hbm, acc_sh.at[pl.ds(r0, STRIPE)])
    pltpu.sync_copy(ones_hbm, ones_v)
    pltpu.sync_copy(dstp_hbm.at[wid], dst_v)
    plsc.subcore_barrier()

    def blk(j, carry):
      pltpu.sync_copy(ones_v, acc_sh.at[dst_v.at[j]], add=True)
      return carry

    lax.fori_loop(0, NB, blk, 0)
    plsc.subcore_barrier()
    pltpu.sync_copy(acc_sh.at[pl.ds(r0, STRIPE)],
                    out_hbm.at[cid, pl.ds(r0, STRIPE)])

  return pl.kernel(
      body,
      out_type=jax.ShapeDtypeStruct((NC, NP, F), jnp.float32),
      mesh=_MESH,
      compiler_params=pltpu.CompilerParams(use_tc_tiling_on_sc=False),
      scratch_types=[
          pltpu.VMEM((BLK, F), jnp.float32),
          pltpu.VMEM((NB, BLK), jnp.int32),
          pltpu.SemaphoreType.DMA,
          pltpu.VMEM_SHARED((NP, F), jnp.float32),
      ],
  )


_RB = 2000


def _dinv_of(deg_blk):
  return 1.0 / jnp.sqrt(deg_blk[0, :, 0:1] + deg_blk[1, :, 0:1] + 1.0)


def _lin_body(x_ref, w_ref, deg_ref, o_ref):
  d = _dinv_of(deg_ref[...])
  o_ref[...] = d * jnp.dot(
      x_ref[...], w_ref[...], preferred_element_type=jnp.float32)


def _lin(x, w, degp):
  di, do = w.shape
  return pl.pallas_call(
      _lin_body,
      grid=(NN // _RB,),
      in_specs=[
          pl.BlockSpec((_RB, di), lambda k: (k, 0)),
          pl.BlockSpec((di, do), lambda k: (0, 0)),
          pl.BlockSpec((2, _RB, 16), lambda k: (0, k, 0)),
      ],
      out_specs=pl.BlockSpec((_RB, do), lambda k: (k, 0)),
      out_shape=jax.ShapeDtypeStruct((NN, do), jnp.float32),
  )(x, w, degp)


def _combine_body(relu, nxt, s_ref, g_ref, deg_ref, b_ref, *rest):
  d = _dinv_of(deg_ref[...])
  s = s_ref[...]
  h = d * (s[0] + s[1] + g_ref[...]) + b_ref[...]
  if relu:
    h = jnp.maximum(h, 0.0)
  if nxt:
    w_ref, o_ref = rest
    o_ref[...] = d * jnp.dot(h, w_ref[...],
                             preferred_element_type=jnp.float32)
  else:
    (o_ref,) = rest
    o_ref[...] = h


def _combine(s, g, degp, b, w=None, relu=True):
  F = g.shape[1]
  specs = [
      pl.BlockSpec((2, _RB, F), lambda k: (0, k, 0)),
      pl.BlockSpec((_RB, F), lambda k: (k, 0)),
      pl.BlockSpec((2, _RB, 16), lambda k: (0, k, 0)),
      pl.BlockSpec((1, F), lambda k: (0, 0)),
  ]
  args = [s, g, degp, b.reshape(1, F)]
  Fo = F
  if w is not None:
    Fo = w.shape[1]
    specs.append(pl.BlockSpec((F, Fo), lambda k: (0, 0)))
    args.append(w)
  return pl.pallas_call(
      functools.partial(_combine_body, relu, w is not None),
      grid=(NN // _RB,),
      in_specs=specs,
      out_specs=pl.BlockSpec((_RB, Fo), lambda k: (k, 0)),
      out_shape=jax.ShapeDtypeStruct((NN, Fo), jnp.float32),
  )(*args)


_KB = 2000
_CB = 3200


def _enc_body(h_ref, we_ref, be_ref, o_ref, acc_ref):
  k = pl.program_id(0)

  @pl.when(k == 0)
  def _():
    acc_ref[...] = jnp.zeros_like(acc_ref)

  acc_ref[...] += jnp.dot(h_ref[0], we_ref[...],
                          preferred_element_type=jnp.float32)

  @pl.when(k == pl.num_programs(0) - 1)
  def _():
    o_ref[...] = acc_ref[...] + be_ref[...]


def _encoder(h3r, we, be):
  L = we.shape[1]
  return pl.pallas_call(
      _enc_body,
      grid=(h3r.shape[0],),
      in_specs=[
          pl.BlockSpec((1, 1, _KB), lambda k: (k, 0, 0)),
          pl.BlockSpec((_KB, L), lambda k: (k, 0)),
          pl.BlockSpec((1, L), lambda k: (0, 0)),
      ],
      out_specs=pl.BlockSpec((1, L), lambda k: (0, 0)),
      out_shape=jax.ShapeDtypeStruct((1, L), jnp.float32),
      scratch_shapes=[pltpu.VMEM((1, L), jnp.float32)],
  )(h3r, we, be.reshape(1, L))


def _dec_body(z_ref, wd_ref, bd_ref, o_ref):
  o_ref[...] = jnp.dot(z_ref[...], wd_ref[...],
                       preferred_element_type=jnp.float32) + bd_ref[...]


def _decoder(z, wd, bd):
  L, M = wd.shape
  return pl.pallas_call(
      _dec_body,
      grid=(M // _CB,),
      in_specs=[
          pl.BlockSpec((1, L), lambda k: (0, 0)),
          pl.BlockSpec((L, _CB), lambda k: (0, k)),
          pl.BlockSpec((1, _CB), lambda k: (0, k)),
      ],
      out_specs=pl.BlockSpec((1, _CB), lambda k: (0, k)),
      out_shape=jax.ShapeDtypeStruct((1, M), jnp.float32),
  )(z, wd, bd.reshape(1, M))


def kernel(x, edge_index, batch_size, batch_index, W1, b1, W2, b2, W3, b3,
           We, be, Wd, bd, W4, b4, W5, b5, W6, b6):
  f32 = jnp.float32
  pad = NW * NB * BLK - EE
  srcp = jnp.concatenate(
      [edge_index[0], jnp.zeros((pad,), jnp.int32)]).reshape(NW, NB, BLK)
  dstp = jnp.concatenate(
      [edge_index[1], jnp.full((pad,), TRASH, jnp.int32)]).reshape(NW, NB, BLK)

  def seg(g):
    F = g.shape[1]
    return _sc_segment_sum(F)(g, srcp, dstp, jnp.zeros((STRIPE, F), f32))

  degp = _sc_degree()(jnp.ones((BLK, 16), f32), dstp,
                      jnp.zeros((STRIPE, 16), f32))

  g1 = _lin(x, W1, degp)
  g2 = _combine(seg(g1), g1, degp, b1, W2)
  g3 = _combine(seg(g2), g2, degp, b2, W3)
  h3 = _combine(seg(g3), g3, degp, b3, relu=False)

  z = _encoder(h3.reshape(NN * 16 // _KB, 1, _KB), We, be)
  h4f = _decoder(z, Wd, bd)
  h4 = h4f.reshape(NN, 16)

  g4 = _lin(h4, W4, degp)
  g5 = _combine(seg(g4), g4, degp, b4, W5)
  W6p = jnp.concatenate([W6, jnp.zeros((W6.shape[0], 15), f32)], axis=1)
  b6p = jnp.concatenate([b6, jnp.zeros((15,), f32)])
  g6 = _combine(seg(g5), g5, degp, b5, W6p)
  out = _combine(seg(g6), g6, degp, b6p, relu=False)
  return out[:, :1].reshape(1, NN)
